# baseline (speedup 1.0000x reference)
import functools

import jax
import jax.numpy as jnp
from jax import lax
from jax.experimental import pallas as pl
from jax.experimental.pallas import tpu as pltpu
from jax.experimental.pallas import tpu_sc as plsc

N_ATOMS = 10000
N_BONDS = 160000
MAX_NB = 16
H = 64


def _block_diag2(w):
    k, n = w.shape
    z = jnp.zeros((k, n), w.dtype)
    return jnp.concatenate(
        [jnp.concatenate([w, z], axis=1), jnp.concatenate([z, w], axis=1)], axis=0
    )


def _bond_in_body(fb_ref, wi_ref, ni_ref):
    ni_ref[...] = jnp.dot(fb_ref[...], wi_ref[...], preferred_element_type=jnp.float32)


def _bond_in(fb2, wi2):
    nb2, d2 = fb2.shape
    blk = 4000
    return pl.pallas_call(
        _bond_in_body,
        grid=(nb2 // blk,),
        in_specs=[
            pl.BlockSpec((blk, d2), lambda i: (i, 0)),
            pl.BlockSpec((d2, 2 * H), lambda i: (0, 0)),
        ],
        out_specs=pl.BlockSpec((blk, 2 * H), lambda i: (i, 0)),
        out_shape=jax.ShapeDtypeStruct((nb2, 2 * H), jnp.float32),
    )(fb2, wi2)


def _bond_update_body(ns_ref, wh_ref, ni_ref, out_ref):
    y = jnp.dot(ns_ref[...], wh_ref[...], preferred_element_type=jnp.float32)
    out_ref[...] = ni_ref[...] + y


def _bond_update(ns2, wh2, ni2):
    nb2 = ns2.shape[0]
    blk = 4000
    return pl.pallas_call(
        _bond_update_body,
        grid=(nb2 // blk,),
        in_specs=[
            pl.BlockSpec((blk, 2 * H), lambda i: (i, 0)),
            pl.BlockSpec((2 * H, 2 * H), lambda i: (0, 0)),
            pl.BlockSpec((blk, 2 * H), lambda i: (i, 0)),
        ],
        out_specs=pl.BlockSpec((blk, 2 * H), lambda i: (i, 0)),
        out_shape=jax.ShapeDtypeStruct((nb2, 2 * H), jnp.float32),
    )(ns2, wh2, ni2)


def _atom_out_body(fa_ref, na_ref, wo1_ref, wo2_ref, bo_ref, out_ref):
    y = jnp.dot(fa_ref[...], wo1_ref[...], preferred_element_type=jnp.float32)
    y = y + jnp.dot(na_ref[...], wo2_ref[...], preferred_element_type=jnp.float32)
    out_ref[...] = y + bo_ref[...]


def _atom_out(fa2, na2, wo1_2, wo2_2, bo2):
    na2_rows, d2 = fa2.shape
    blk = 1000
    return pl.pallas_call(
        _atom_out_body,
        grid=(na2_rows // blk,),
        in_specs=[
            pl.BlockSpec((blk, d2), lambda i: (i, 0)),
            pl.BlockSpec((blk, 2 * H), lambda i: (i, 0)),
            pl.BlockSpec((d2, 2 * H), lambda i: (0, 0)),
            pl.BlockSpec((2 * H, 2 * H), lambda i: (0, 0)),
            pl.BlockSpec((1, 2 * H), lambda i: (0, 0)),
        ],
        out_specs=pl.BlockSpec((blk, 2 * H), lambda i: (i, 0)),
        out_shape=jax.ShapeDtypeStruct((na2_rows, 2 * H), jnp.float32),
    )(fa2, na2, wo1_2, wo2_2, bo2)


def _mol_head_body(a3_ref, wmh_ref, bmh_ref, wmo_ref, bmo_ref, out_ref):
    s2 = jnp.sum(a3_ref[...], axis=1)
    s = s2[:, :H] + s2[:, H:]
    h = jnp.dot(s, wmh_ref[...], preferred_element_type=jnp.float32) + bmh_ref[...]
    h = jnp.maximum(h, 0.0)
    out_ref[...] = jnp.dot(h, wmo_ref[...], preferred_element_type=jnp.float32) + bmo_ref[...]


def _mol_head(atom_h3, w_mh, b_mh, w_mo, b_mo):
    nm, seg2, _ = atom_h3.shape
    fh = w_mh.shape[1]
    nl = w_mo.shape[1]
    return pl.pallas_call(
        _mol_head_body,
        grid=(1,),
        in_specs=[
            pl.BlockSpec((nm, seg2, 2 * H), lambda i: (0, 0, 0)),
            pl.BlockSpec((H, fh), lambda i: (0, 0)),
            pl.BlockSpec((1, fh), lambda i: (0, 0)),
            pl.BlockSpec((fh, nl), lambda i: (0, 0)),
            pl.BlockSpec((1, nl), lambda i: (0, 0)),
        ],
        out_specs=pl.BlockSpec((nm, nl), lambda i: (0, 0)),
        out_shape=jax.ShapeDtypeStruct((nm, nl), jnp.float32),
    )(atom_h3, w_mh, b_mh.reshape(1, fh), w_mo, b_mo.reshape(1, nl))


_B = 32
_RPB = _B * MAX_NB
_NDMA = _RPB // 128


def _gather_sum_relu_sc(table, idx, m_rows, nb_fast, nb_slow, nb_slow_extra):
    assert 16 * (nb_fast + nb_slow) + nb_slow_extra == m_rows // _B
    assert m_rows % _B == 0
    idx2d = idx.reshape(m_rows * MAX_NB // 128, 128)

    mesh = plsc.VectorSubcoreMesh(core_axis_name="c", subcore_axis_name="s")

    @functools.partial(
        pl.kernel,
        mesh=mesh,
        compiler_params=pltpu.CompilerParams(
            use_tc_tiling_on_sc=False, needs_layout_passes=False
        ),
        out_type=jax.ShapeDtypeStruct((m_rows, H), jnp.float32),
        scratch_types=[
            pltpu.VMEM((2, _NDMA, 128), jnp.int32),
            pltpu.VMEM((2, _RPB, H), jnp.float32),
            pltpu.VMEM((2, _B, H), jnp.float32),
            pltpu.SemaphoreType.DMA,
            pltpu.SemaphoreType.DMA,
            pltpu.SemaphoreType.DMA,
            pltpu.SemaphoreType.DMA,
        ],
    )
    def gsum(table_hbm, idx_hbm, out_hbm, idx_v, rows_v, out_v, sg0, sg1, so0, so1):
        cid = lax.axis_index("c")
        sid = lax.axis_index("s")
        nb = jnp.where(
            cid == 0,
            nb_fast,
            nb_slow + jnp.where(sid < nb_slow_extra, 1, 0),
        )
        base_b = jnp.where(
            cid == 0,
            sid * nb_fast,
            16 * nb_fast + sid * nb_slow + jnp.minimum(sid, nb_slow_extra),
        )
        npairs = nb // 2
        tail = nb - 2 * npairs

        def fire(bg, slot, sem):
            pltpu.sync_copy(idx_hbm.at[pl.ds(bg * _NDMA, _NDMA)], idx_v.at[slot])
            for d in range(_NDMA):
                pltpu.async_copy(
                    table_hbm.at[idx_v.at[slot].at[d]],
                    rows_v.at[slot].at[pl.ds(d * 128, 128)],
                    sem,
                )

        def wait_gather(slot, sem):
            pltpu.make_async_copy(
                table_hbm.at[pl.ds(0, _RPB)], rows_v.at[slot], sem
            ).wait()

        def reduce(slot):
            zero = jnp.zeros((16,), jnp.float32)

            def row(i, _):
                r0 = i * MAX_NB
                for c in range(H // 16):
                    sl = pl.ds(c * 16, 16)
                    acc = jnp.maximum(rows_v[slot, r0, sl], zero)
                    for j in range(1, MAX_NB):
                        acc = acc + jnp.maximum(rows_v[slot, r0 + j, sl], zero)
                    out_v[slot, i, sl] = acc
                return 0

            lax.fori_loop(0, _B, row, 0, unroll=False)

        def store(bg, slot, sem):
            pltpu.async_copy(out_v.at[slot], out_hbm.at[pl.ds(bg * _B, _B)], sem)

        def wait_store(slot, sem):
            pltpu.make_async_copy(
                out_v.at[slot], out_hbm.at[pl.ds(0, _B)], sem
            ).wait()

        fire(base_b, 0, sg0)

        def pair(k, _):
            b0 = base_b + 2 * k
            fire(b0 + 1, 1, sg1)
            wait_gather(0, sg0)

            @pl.when(k > 0)
            def _():
                wait_store(0, so0)

            reduce(0)
            store(b0, 0, so0)

            @pl.when(2 * k + 2 < nb)
            def _():
                fire(b0 + 2, 0, sg0)

            wait_gather(1, sg1)

            @pl.when(k > 0)
            def _():
                wait_store(1, so1)

            reduce(1)
            store(b0 + 1, 1, so1)
            return 0

        lax.fori_loop(0, npairs, pair, 0, unroll=False)

        @pl.when(tail == 1)
        def _():
            wait_gather(0, sg0)

            @pl.when(npairs > 0)
            def _():
                wait_store(0, so0)

            reduce(0)
            store(base_b + nb - 1, 0, so0)

        wait_store(0, so0)

        @pl.when(npairs > 0)
        def _():
            wait_store(1, so1)

    return gsum(table, idx2d)


def kernel(fatoms, fbonds, agraph, bgraph, scope, w_i, w_h, w_o, b_o, w_mh, b_mh, w_mo, b_mo):
    del scope
    d = fatoms.shape[1]

    fb2 = jnp.reshape(fbonds, (N_BONDS // 2, 2 * (d + 16)))
    ni2 = _bond_in(fb2, _block_diag2(w_i))
    ni = jnp.reshape(ni2, (N_BONDS, H))

    nei_sum = _gather_sum_relu_sc(ni, bgraph, N_BONDS, 156, 156, 8)
    ns2 = jnp.reshape(nei_sum, (N_BONDS // 2, 2 * H))
    h2 = _bond_update(ns2, _block_diag2(w_h), ni2)
    msg_2 = jnp.reshape(h2, (N_BONDS, H))

    a_pad = 10240
    agraph_p = jnp.concatenate(
        [agraph, jnp.zeros((a_pad - N_ATOMS, MAX_NB), jnp.int32)], axis=0
    )
    nei_atom = _gather_sum_relu_sc(msg_2, agraph_p, a_pad, 10, 10, 0)
    na2 = jnp.reshape(nei_atom, (a_pad // 2, 2 * H))[: N_ATOMS // 2]

    fa2 = jnp.reshape(fatoms, (N_ATOMS // 2, 2 * d))
    ah2 = _atom_out(
        fa2,
        na2,
        _block_diag2(w_o[:d]),
        _block_diag2(w_o[d:]),
        jnp.concatenate([b_o, b_o]).reshape(1, 2 * H),
    )
    atom_h = jnp.reshape(ah2, (N_ATOMS, H))
    mol_o = _mol_head(ah2.reshape(500, 10, 2 * H), w_mh, b_mh, w_mo, b_mo)
    return (atom_h, mol_o)


# speedup vs baseline: 1.3184x; 1.3184x over previous
import functools

import jax
import jax.numpy as jnp
from jax import lax
from jax.experimental import pallas as pl
from jax.experimental.pallas import tpu as pltpu
from jax.experimental.pallas import tpu_sc as plsc

N_ATOMS = 10000
N_BONDS = 160000
MAX_NB = 16
H = 64


def _block_diag2(w):
    k, n = w.shape
    z = jnp.zeros((k, n), w.dtype)
    return jnp.concatenate(
        [jnp.concatenate([w, z], axis=1), jnp.concatenate([z, w], axis=1)], axis=0
    )


def _bond_in_body(fb_ref, wi_ref, ni_ref):
    ni_ref[...] = jnp.dot(fb_ref[...], wi_ref[...], preferred_element_type=jnp.float32)


def _bond_in(fb2, wi2):
    nb2, d2 = fb2.shape
    blk = 4000
    return pl.pallas_call(
        _bond_in_body,
        grid=(nb2 // blk,),
        in_specs=[
            pl.BlockSpec((blk, d2), lambda i: (i, 0)),
            pl.BlockSpec((d2, 2 * H), lambda i: (0, 0)),
        ],
        out_specs=pl.BlockSpec((blk, 2 * H), lambda i: (i, 0)),
        out_shape=jax.ShapeDtypeStruct((nb2, 2 * H), jnp.float32),
    )(fb2, wi2)


def _bond_update_body(ns_ref, wh_ref, ni_ref, out_ref):
    y = jnp.dot(ns_ref[...], wh_ref[...], preferred_element_type=jnp.float32)
    out_ref[...] = ni_ref[...] + y


def _bond_update(ns2, wh2, ni2):
    nb2 = ns2.shape[0]
    blk = 4000
    return pl.pallas_call(
        _bond_update_body,
        grid=(nb2 // blk,),
        in_specs=[
            pl.BlockSpec((blk, 2 * H), lambda i: (i, 0)),
            pl.BlockSpec((2 * H, 2 * H), lambda i: (0, 0)),
            pl.BlockSpec((blk, 2 * H), lambda i: (i, 0)),
        ],
        out_specs=pl.BlockSpec((blk, 2 * H), lambda i: (i, 0)),
        out_shape=jax.ShapeDtypeStruct((nb2, 2 * H), jnp.float32),
    )(ns2, wh2, ni2)


def _atom_out_body(fa_ref, na_ref, wo1_ref, wo2_ref, bo_ref, out_ref):
    y = jnp.dot(fa_ref[...], wo1_ref[...], preferred_element_type=jnp.float32)
    y = y + jnp.dot(na_ref[...], wo2_ref[...], preferred_element_type=jnp.float32)
    out_ref[...] = y + bo_ref[...]


def _atom_out(fa2, na2, wo1_2, wo2_2, bo2):
    na2_rows, d2 = fa2.shape
    blk = 1000
    return pl.pallas_call(
        _atom_out_body,
        grid=(na2_rows // blk,),
        in_specs=[
            pl.BlockSpec((blk, d2), lambda i: (i, 0)),
            pl.BlockSpec((blk, 2 * H), lambda i: (i, 0)),
            pl.BlockSpec((d2, 2 * H), lambda i: (0, 0)),
            pl.BlockSpec((2 * H, 2 * H), lambda i: (0, 0)),
            pl.BlockSpec((1, 2 * H), lambda i: (0, 0)),
        ],
        out_specs=pl.BlockSpec((blk, 2 * H), lambda i: (i, 0)),
        out_shape=jax.ShapeDtypeStruct((na2_rows, 2 * H), jnp.float32),
    )(fa2, na2, wo1_2, wo2_2, bo2)


def _mol_head_body(a3_ref, wmh_ref, bmh_ref, wmo_ref, bmo_ref, out_ref):
    s2 = jnp.sum(a3_ref[...], axis=1)
    s = s2[:, :H] + s2[:, H:]
    h = jnp.dot(s, wmh_ref[...], preferred_element_type=jnp.float32) + bmh_ref[...]
    h = jnp.maximum(h, 0.0)
    out_ref[...] = jnp.dot(h, wmo_ref[...], preferred_element_type=jnp.float32) + bmo_ref[...]


def _mol_head(atom_h3, w_mh, b_mh, w_mo, b_mo):
    nm, seg2, _ = atom_h3.shape
    fh = w_mh.shape[1]
    nl = w_mo.shape[1]
    return pl.pallas_call(
        _mol_head_body,
        grid=(1,),
        in_specs=[
            pl.BlockSpec((nm, seg2, 2 * H), lambda i: (0, 0, 0)),
            pl.BlockSpec((H, fh), lambda i: (0, 0)),
            pl.BlockSpec((1, fh), lambda i: (0, 0)),
            pl.BlockSpec((fh, nl), lambda i: (0, 0)),
            pl.BlockSpec((1, nl), lambda i: (0, 0)),
        ],
        out_specs=pl.BlockSpec((nm, nl), lambda i: (0, 0)),
        out_shape=jax.ShapeDtypeStruct((nm, nl), jnp.float32),
    )(atom_h3, w_mh, b_mh.reshape(1, fh), w_mo, b_mo.reshape(1, nl))


_B = 32
_RPB = _B * MAX_NB
_NDMA = _RPB // 128


def _gather_sum_relu_sc(table, idxt, m_rows, nb_fast, nb_slow, nb_slow_extra):
    assert 16 * (nb_fast + nb_slow) + nb_slow_extra == m_rows // _B
    assert m_rows % _B == 0
    assert idxt.shape == (MAX_NB, m_rows)

    mesh = plsc.VectorSubcoreMesh(core_axis_name="c", subcore_axis_name="s")

    @functools.partial(
        pl.kernel,
        mesh=mesh,
        compiler_params=pltpu.CompilerParams(
            use_tc_tiling_on_sc=False, needs_layout_passes=False
        ),
        out_type=jax.ShapeDtypeStruct((m_rows, H), jnp.float32),
        scratch_types=[
            pltpu.VMEM((2, MAX_NB, _B), jnp.int32),
            pltpu.VMEM((2, _RPB, H), jnp.float32),
            pltpu.VMEM((2, _B, H), jnp.float32),
            pltpu.SemaphoreType.DMA,
            pltpu.SemaphoreType.DMA,
            pltpu.SemaphoreType.DMA,
            pltpu.SemaphoreType.DMA,
        ],
    )
    def gsum(table_hbm, idx_hbm, out_hbm, idx_v, rows_v, out_v, sg0, sg1, so0, so1):
        cid = lax.axis_index("c")
        sid = lax.axis_index("s")
        nb = jnp.where(
            cid == 0,
            nb_fast,
            nb_slow + jnp.where(sid < nb_slow_extra, 1, 0),
        )
        base_b = jnp.where(
            cid == 0,
            sid * nb_fast,
            16 * nb_fast + sid * nb_slow + jnp.minimum(sid, nb_slow_extra),
        )
        npairs = nb // 2
        tail = nb - 2 * npairs

        def fire(bg, slot, sem):
            pltpu.sync_copy(
                idx_hbm.at[:, pl.ds(bg * _B, _B)], idx_v.at[slot]
            )
            for j in range(MAX_NB):
                pltpu.async_copy(
                    table_hbm.at[idx_v.at[slot].at[j]],
                    rows_v.at[slot].at[pl.ds(j * _B, _B)],
                    sem,
                )

        def wait_gather(slot, sem):
            pltpu.make_async_copy(
                table_hbm.at[pl.ds(0, _RPB)], rows_v.at[slot], sem
            ).wait()

        def reduce(slot):
            zero = jnp.zeros((16,), jnp.float32)

            def row(i, _):
                for c in range(H // 16):
                    sl = pl.ds(c * 16, 16)
                    acc = jnp.maximum(rows_v[slot, i, sl], zero)
                    for j in range(1, MAX_NB):
                        acc = acc + jnp.maximum(rows_v[slot, j * _B + i, sl], zero)
                    out_v[slot, i, sl] = acc
                return 0

            lax.fori_loop(0, _B, row, 0, unroll=False)

        def store(bg, slot, sem):
            pltpu.async_copy(out_v.at[slot], out_hbm.at[pl.ds(bg * _B, _B)], sem)

        def wait_store(slot, sem):
            pltpu.make_async_copy(
                out_v.at[slot], out_hbm.at[pl.ds(0, _B)], sem
            ).wait()

        fire(base_b, 0, sg0)

        def pair(k, _):
            b0 = base_b + 2 * k
            fire(b0 + 1, 1, sg1)
            wait_gather(0, sg0)

            @pl.when(k > 0)
            def _():
                wait_store(0, so0)

            reduce(0)
            store(b0, 0, so0)

            @pl.when(2 * k + 2 < nb)
            def _():
                fire(b0 + 2, 0, sg0)

            wait_gather(1, sg1)

            @pl.when(k > 0)
            def _():
                wait_store(1, so1)

            reduce(1)
            store(b0 + 1, 1, so1)
            return 0

        lax.fori_loop(0, npairs, pair, 0, unroll=False)

        @pl.when(tail == 1)
        def _():
            wait_gather(0, sg0)

            @pl.when(npairs > 0)
            def _():
                wait_store(0, so0)

            reduce(0)
            store(base_b + nb - 1, 0, so0)

        wait_store(0, so0)

        @pl.when(npairs > 0)
        def _():
            wait_store(1, so1)

    return gsum(table, idxt)


def kernel(fatoms, fbonds, agraph, bgraph, scope, w_i, w_h, w_o, b_o, w_mh, b_mh, w_mo, b_mo):
    del scope
    d = fatoms.shape[1]

    fb2 = jnp.reshape(fbonds, (N_BONDS // 2, 2 * (d + 16)))
    ni2 = _bond_in(fb2, _block_diag2(w_i))
    ni = jnp.reshape(ni2, (N_BONDS, H))

    nei_sum = _gather_sum_relu_sc(ni, bgraph.T, N_BONDS, 156, 156, 8)
    ns2 = jnp.reshape(nei_sum, (N_BONDS // 2, 2 * H))
    h2 = _bond_update(ns2, _block_diag2(w_h), ni2)
    msg_2 = jnp.reshape(h2, (N_BONDS, H))

    a_pad = 10240
    agraph_pt = jnp.concatenate(
        [agraph.T, jnp.zeros((MAX_NB, a_pad - N_ATOMS), jnp.int32)], axis=1
    )
    nei_atom = _gather_sum_relu_sc(msg_2, agraph_pt, a_pad, 14, 6, 0)
    na2 = jnp.reshape(nei_atom, (a_pad // 2, 2 * H))[: N_ATOMS // 2]

    fa2 = jnp.reshape(fatoms, (N_ATOMS // 2, 2 * d))
    ah2 = _atom_out(
        fa2,
        na2,
        _block_diag2(w_o[:d]),
        _block_diag2(w_o[d:]),
        jnp.concatenate([b_o, b_o]).reshape(1, 2 * H),
    )
    atom_h = jnp.reshape(ah2, (N_ATOMS, H))
    mol_o = _mol_head(ah2.reshape(500, 10, 2 * H), w_mh, b_mh, w_mo, b_mo)
    return (atom_h, mol_o)


# speedup vs baseline: 1.3205x; 1.0016x over previous
import functools

import jax
import jax.numpy as jnp
from jax import lax
from jax.experimental import pallas as pl
from jax.experimental.pallas import tpu as pltpu
from jax.experimental.pallas import tpu_sc as plsc

N_ATOMS = 10000
N_BONDS = 160000
MAX_NB = 16
H = 64


def _block_diag2(w):
    k, n = w.shape
    z = jnp.zeros((k, n), w.dtype)
    return jnp.concatenate(
        [jnp.concatenate([w, z], axis=1), jnp.concatenate([z, w], axis=1)], axis=0
    )


def _bond_in_body(fb_ref, wi_ref, ni_ref):
    ni_ref[...] = jnp.dot(fb_ref[...], wi_ref[...], preferred_element_type=jnp.float32)


def _bond_in(fb2, wi2):
    nb2, d2 = fb2.shape
    blk = 4000
    return pl.pallas_call(
        _bond_in_body,
        grid=(nb2 // blk,),
        in_specs=[
            pl.BlockSpec((blk, d2), lambda i: (i, 0)),
            pl.BlockSpec((d2, 2 * H), lambda i: (0, 0)),
        ],
        out_specs=pl.BlockSpec((blk, 2 * H), lambda i: (i, 0)),
        out_shape=jax.ShapeDtypeStruct((nb2, 2 * H), jnp.float32),
    )(fb2, wi2)


def _bond_update_body(ns_ref, wh_ref, ni_ref, out_ref):
    y = jnp.dot(ns_ref[...], wh_ref[...], preferred_element_type=jnp.float32)
    out_ref[...] = ni_ref[...] + y


def _bond_update(ns2, wh2, ni2):
    nb2 = ns2.shape[0]
    blk = 4000
    return pl.pallas_call(
        _bond_update_body,
        grid=(nb2 // blk,),
        in_specs=[
            pl.BlockSpec((blk, 2 * H), lambda i: (i, 0)),
            pl.BlockSpec((2 * H, 2 * H), lambda i: (0, 0)),
            pl.BlockSpec((blk, 2 * H), lambda i: (i, 0)),
        ],
        out_specs=pl.BlockSpec((blk, 2 * H), lambda i: (i, 0)),
        out_shape=jax.ShapeDtypeStruct((nb2, 2 * H), jnp.float32),
    )(ns2, wh2, ni2)


def _atom_out_body(fa_ref, na_ref, wo1_ref, wo2_ref, bo_ref, out_ref):
    y = jnp.dot(fa_ref[...], wo1_ref[...], preferred_element_type=jnp.float32)
    y = y + jnp.dot(na_ref[...], wo2_ref[...], preferred_element_type=jnp.float32)
    out_ref[...] = y + bo_ref[...]


def _atom_out(fa2, na2, wo1_2, wo2_2, bo2):
    na2_rows, d2 = fa2.shape
    blk = 1000
    return pl.pallas_call(
        _atom_out_body,
        grid=(na2_rows // blk,),
        in_specs=[
            pl.BlockSpec((blk, d2), lambda i: (i, 0)),
            pl.BlockSpec((blk, 2 * H), lambda i: (i, 0)),
            pl.BlockSpec((d2, 2 * H), lambda i: (0, 0)),
            pl.BlockSpec((2 * H, 2 * H), lambda i: (0, 0)),
            pl.BlockSpec((1, 2 * H), lambda i: (0, 0)),
        ],
        out_specs=pl.BlockSpec((blk, 2 * H), lambda i: (i, 0)),
        out_shape=jax.ShapeDtypeStruct((na2_rows, 2 * H), jnp.float32),
    )(fa2, na2, wo1_2, wo2_2, bo2)


def _mol_head_body(a3_ref, wmh_ref, bmh_ref, wmo_ref, bmo_ref, out_ref):
    s2 = jnp.sum(a3_ref[...], axis=1)
    s = s2[:, :H] + s2[:, H:]
    h = jnp.dot(s, wmh_ref[...], preferred_element_type=jnp.float32) + bmh_ref[...]
    h = jnp.maximum(h, 0.0)
    out_ref[...] = jnp.dot(h, wmo_ref[...], preferred_element_type=jnp.float32) + bmo_ref[...]


def _mol_head(atom_h3, w_mh, b_mh, w_mo, b_mo):
    nm, seg2, _ = atom_h3.shape
    fh = w_mh.shape[1]
    nl = w_mo.shape[1]
    return pl.pallas_call(
        _mol_head_body,
        grid=(1,),
        in_specs=[
            pl.BlockSpec((nm, seg2, 2 * H), lambda i: (0, 0, 0)),
            pl.BlockSpec((H, fh), lambda i: (0, 0)),
            pl.BlockSpec((1, fh), lambda i: (0, 0)),
            pl.BlockSpec((fh, nl), lambda i: (0, 0)),
            pl.BlockSpec((1, nl), lambda i: (0, 0)),
        ],
        out_specs=pl.BlockSpec((nm, nl), lambda i: (0, 0)),
        out_shape=jax.ShapeDtypeStruct((nm, nl), jnp.float32),
    )(atom_h3, w_mh, b_mh.reshape(1, fh), w_mo, b_mo.reshape(1, nl))


_B = 40
_RPB = _B * MAX_NB


def _gather_sum_relu_sc(table, idxt, m_rows, nb_fast, nb_slow, nb_slow_extra):
    assert 16 * (nb_fast + nb_slow) + nb_slow_extra == m_rows // _B
    assert m_rows % _B == 0
    assert idxt.shape == (MAX_NB, m_rows)

    mesh = plsc.VectorSubcoreMesh(core_axis_name="c", subcore_axis_name="s")

    @functools.partial(
        pl.kernel,
        mesh=mesh,
        compiler_params=pltpu.CompilerParams(
            use_tc_tiling_on_sc=False, needs_layout_passes=False
        ),
        out_type=jax.ShapeDtypeStruct((m_rows, H), jnp.float32),
        scratch_types=[
            pltpu.VMEM((2, MAX_NB, _B), jnp.int32),
            pltpu.VMEM((2, _RPB, H), jnp.float32),
            pltpu.VMEM((2, _B, H), jnp.float32),
            pltpu.SemaphoreType.DMA,
            pltpu.SemaphoreType.DMA,
            pltpu.SemaphoreType.DMA,
            pltpu.SemaphoreType.DMA,
        ],
    )
    def gsum(table_hbm, idx_hbm, out_hbm, idx_v, rows_v, out_v, sg0, sg1, so0, so1):
        cid = lax.axis_index("c")
        sid = lax.axis_index("s")
        nb = jnp.where(
            cid == 0,
            nb_fast,
            nb_slow + jnp.where(sid < nb_slow_extra, 1, 0),
        )
        base_b = jnp.where(
            cid == 0,
            sid * nb_fast,
            16 * nb_fast + sid * nb_slow + jnp.minimum(sid, nb_slow_extra),
        )
        npairs = nb // 2
        tail = nb - 2 * npairs

        def fire(bg, slot, sem):
            pltpu.sync_copy(
                idx_hbm.at[:, pl.ds(bg * _B, _B)], idx_v.at[slot]
            )
            for j in range(MAX_NB):
                pltpu.async_copy(
                    table_hbm.at[idx_v.at[slot].at[j]],
                    rows_v.at[slot].at[pl.ds(j * _B, _B)],
                    sem,
                )

        def wait_gather(slot, sem):
            pltpu.make_async_copy(
                table_hbm.at[pl.ds(0, _RPB)], rows_v.at[slot], sem
            ).wait()

        def reduce(slot):
            zero = jnp.zeros((16,), jnp.float32)

            def row(i, _):
                for c in range(H // 16):
                    sl = pl.ds(c * 16, 16)
                    acc = jnp.maximum(rows_v[slot, i, sl], zero)
                    for j in range(1, MAX_NB):
                        acc = acc + jnp.maximum(rows_v[slot, j * _B + i, sl], zero)
                    out_v[slot, i, sl] = acc
                return 0

            lax.fori_loop(0, _B, row, 0, unroll=False)

        def store(bg, slot, sem):
            pltpu.async_copy(out_v.at[slot], out_hbm.at[pl.ds(bg * _B, _B)], sem)

        def wait_store(slot, sem):
            pltpu.make_async_copy(
                out_v.at[slot], out_hbm.at[pl.ds(0, _B)], sem
            ).wait()

        @pl.when(nb > 0)
        def _():
            fire(base_b, 0, sg0)

        def pair(k, _):
            b0 = base_b + 2 * k
            fire(b0 + 1, 1, sg1)
            wait_gather(0, sg0)

            @pl.when(k > 0)
            def _():
                wait_store(0, so0)

            reduce(0)
            store(b0, 0, so0)

            @pl.when(2 * k + 2 < nb)
            def _():
                fire(b0 + 2, 0, sg0)

            wait_gather(1, sg1)

            @pl.when(k > 0)
            def _():
                wait_store(1, so1)

            reduce(1)
            store(b0 + 1, 1, so1)
            return 0

        lax.fori_loop(0, npairs, pair, 0, unroll=False)

        @pl.when(tail == 1)
        def _():
            wait_gather(0, sg0)

            @pl.when(npairs > 0)
            def _():
                wait_store(0, so0)

            reduce(0)
            store(base_b + nb - 1, 0, so0)

        @pl.when(nb > 0)
        def _():
            wait_store(0, so0)

        @pl.when(npairs > 0)
        def _():
            wait_store(1, so1)

    return gsum(table, idxt)


def kernel(fatoms, fbonds, agraph, bgraph, scope, w_i, w_h, w_o, b_o, w_mh, b_mh, w_mo, b_mo):
    del scope
    d = fatoms.shape[1]

    fb2 = jnp.reshape(fbonds, (N_BONDS // 2, 2 * (d + 16)))
    ni2 = _bond_in(fb2, _block_diag2(w_i))
    ni = jnp.reshape(ni2, (N_BONDS, H))

    nei_sum = _gather_sum_relu_sc(ni, bgraph.T, N_BONDS, 125, 125, 0)
    ns2 = jnp.reshape(nei_sum, (N_BONDS // 2, 2 * H))
    h2 = _bond_update(ns2, _block_diag2(w_h), ni2)
    msg_2 = jnp.reshape(h2, (N_BONDS, H))

    a_pad = 10240
    agraph_pt = jnp.concatenate(
        [agraph.T, jnp.zeros((MAX_NB, a_pad - N_ATOMS), jnp.int32)], axis=1
    )
    nei_atom = _gather_sum_relu_sc(msg_2, agraph_pt, a_pad, 16, 0, 0)
    na2 = jnp.reshape(nei_atom, (a_pad // 2, 2 * H))[: N_ATOMS // 2]

    fa2 = jnp.reshape(fatoms, (N_ATOMS // 2, 2 * d))
    ah2 = _atom_out(
        fa2,
        na2,
        _block_diag2(w_o[:d]),
        _block_diag2(w_o[d:]),
        jnp.concatenate([b_o, b_o]).reshape(1, 2 * H),
    )
    atom_h = jnp.reshape(ah2, (N_ATOMS, H))
    mol_o = _mol_head(ah2.reshape(500, 10, 2 * H), w_mh, b_mh, w_mo, b_mo)
    return (atom_h, mol_o)


# speedup vs baseline: 1.5344x; 1.1620x over previous
import functools

import jax
import jax.numpy as jnp
from jax import lax
from jax.experimental import pallas as pl
from jax.experimental.pallas import tpu as pltpu
from jax.experimental.pallas import tpu_sc as plsc

N_ATOMS = 10000
N_BONDS = 160000
MAX_NB = 16
H = 64


def _block_diag2(w):
    k, n = w.shape
    z = jnp.zeros((k, n), w.dtype)
    return jnp.concatenate(
        [jnp.concatenate([w, z], axis=1), jnp.concatenate([z, w], axis=1)], axis=0
    )


def _bond_in_body(fb_ref, wi_ref, ni_ref):
    ni_ref[...] = jnp.dot(fb_ref[...], wi_ref[...], preferred_element_type=jnp.float32)


def _bond_in(fb2, wi2):
    nb2, d2 = fb2.shape
    blk = 4000
    return pl.pallas_call(
        _bond_in_body,
        grid=(nb2 // blk,),
        in_specs=[
            pl.BlockSpec((blk, d2), lambda i: (i, 0)),
            pl.BlockSpec((d2, 2 * H), lambda i: (0, 0)),
        ],
        out_specs=pl.BlockSpec((blk, 2 * H), lambda i: (i, 0)),
        out_shape=jax.ShapeDtypeStruct((nb2, 2 * H), jnp.float32),
    )(fb2, wi2)


def _bond_update_body(ns_ref, wh_ref, ni_ref, out_ref):
    y = jnp.dot(ns_ref[...], wh_ref[...], preferred_element_type=jnp.float32)
    out_ref[...] = ni_ref[...] + y


def _bond_update(ns2, wh2, ni2):
    nb2 = ns2.shape[0]
    blk = 4000
    return pl.pallas_call(
        _bond_update_body,
        grid=(nb2 // blk,),
        in_specs=[
            pl.BlockSpec((blk, 2 * H), lambda i: (i, 0)),
            pl.BlockSpec((2 * H, 2 * H), lambda i: (0, 0)),
            pl.BlockSpec((blk, 2 * H), lambda i: (i, 0)),
        ],
        out_specs=pl.BlockSpec((blk, 2 * H), lambda i: (i, 0)),
        out_shape=jax.ShapeDtypeStruct((nb2, 2 * H), jnp.float32),
    )(ns2, wh2, ni2)


def _atom_out_body(fa_ref, na_ref, wo1_ref, wo2_ref, bo_ref, out_ref):
    y = jnp.dot(fa_ref[...], wo1_ref[...], preferred_element_type=jnp.float32)
    y = y + jnp.dot(na_ref[...], wo2_ref[...], preferred_element_type=jnp.float32)
    out_ref[...] = y + bo_ref[...]


def _atom_out(fa2, na2, wo1_2, wo2_2, bo2):
    na2_rows, d2 = fa2.shape
    blk = 1000
    return pl.pallas_call(
        _atom_out_body,
        grid=(na2_rows // blk,),
        in_specs=[
            pl.BlockSpec((blk, d2), lambda i: (i, 0)),
            pl.BlockSpec((blk, 2 * H), lambda i: (i, 0)),
            pl.BlockSpec((d2, 2 * H), lambda i: (0, 0)),
            pl.BlockSpec((2 * H, 2 * H), lambda i: (0, 0)),
            pl.BlockSpec((1, 2 * H), lambda i: (0, 0)),
        ],
        out_specs=pl.BlockSpec((blk, 2 * H), lambda i: (i, 0)),
        out_shape=jax.ShapeDtypeStruct((na2_rows, 2 * H), jnp.float32),
    )(fa2, na2, wo1_2, wo2_2, bo2)


def _mol_head_body(a3_ref, wmh_ref, bmh_ref, wmo_ref, bmo_ref, out_ref):
    s2 = jnp.sum(a3_ref[...], axis=1)
    s = s2[:, :H] + s2[:, H:]
    h = jnp.dot(s, wmh_ref[...], preferred_element_type=jnp.float32) + bmh_ref[...]
    h = jnp.maximum(h, 0.0)
    out_ref[...] = jnp.dot(h, wmo_ref[...], preferred_element_type=jnp.float32) + bmo_ref[...]


def _mol_head(atom_h3, w_mh, b_mh, w_mo, b_mo):
    nm, seg2, _ = atom_h3.shape
    fh = w_mh.shape[1]
    nl = w_mo.shape[1]
    return pl.pallas_call(
        _mol_head_body,
        grid=(1,),
        in_specs=[
            pl.BlockSpec((nm, seg2, 2 * H), lambda i: (0, 0, 0)),
            pl.BlockSpec((H, fh), lambda i: (0, 0)),
            pl.BlockSpec((1, fh), lambda i: (0, 0)),
            pl.BlockSpec((fh, nl), lambda i: (0, 0)),
            pl.BlockSpec((1, nl), lambda i: (0, 0)),
        ],
        out_specs=pl.BlockSpec((nm, nl), lambda i: (0, 0)),
        out_shape=jax.ShapeDtypeStruct((nm, nl), jnp.float32),
    )(atom_h3, w_mh, b_mh.reshape(1, fh), w_mo, b_mo.reshape(1, nl))


_B = 40
_RPB = _B * MAX_NB


def _gather_sum_relu_sc(table, idxt, m_rows, nb_fast, nb_fast_extra, nb_slow, nb_slow_extra):
    assert (
        16 * (nb_fast + nb_slow) + nb_fast_extra + nb_slow_extra == m_rows // _B
    )
    assert m_rows % _B == 0
    assert idxt.shape == (MAX_NB, m_rows)

    mesh = plsc.VectorSubcoreMesh(core_axis_name="c", subcore_axis_name="s")

    @functools.partial(
        pl.kernel,
        mesh=mesh,
        compiler_params=pltpu.CompilerParams(
            use_tc_tiling_on_sc=False, needs_layout_passes=False
        ),
        out_type=jax.ShapeDtypeStruct((m_rows, H), jnp.float32),
        scratch_types=[
            pltpu.VMEM((2, MAX_NB, _B), jnp.int32),
            pltpu.VMEM((2, _RPB, H), jnp.float32),
            pltpu.VMEM((2, _B, H), jnp.float32),
            pltpu.SemaphoreType.DMA,
            pltpu.SemaphoreType.DMA,
            pltpu.SemaphoreType.DMA,
            pltpu.SemaphoreType.DMA,
        ],
    )
    def gsum(table_hbm, idx_hbm, out_hbm, idx_v, rows_v, out_v, sg0, sg1, so0, so1):
        cid = lax.axis_index("c")
        sid = lax.axis_index("s")
        nb = jnp.where(
            cid == 0,
            nb_fast + jnp.where(sid < nb_fast_extra, 1, 0),
            nb_slow + jnp.where(sid < nb_slow_extra, 1, 0),
        )
        base_b = jnp.where(
            cid == 0,
            sid * nb_fast + jnp.minimum(sid, nb_fast_extra),
            16 * nb_fast
            + nb_fast_extra
            + sid * nb_slow
            + jnp.minimum(sid, nb_slow_extra),
        )
        npairs = nb // 2
        tail = nb - 2 * npairs

        def fire(bg, slot, sem):
            pltpu.sync_copy(
                idx_hbm.at[:, pl.ds(bg * _B, _B)], idx_v.at[slot]
            )
            for j in range(MAX_NB):
                pltpu.async_copy(
                    table_hbm.at[idx_v.at[slot].at[j]],
                    rows_v.at[slot].at[pl.ds(j * _B, _B)],
                    sem,
                )

        def wait_gather(slot, sem):
            pltpu.make_async_copy(
                table_hbm.at[pl.ds(0, _RPB)], rows_v.at[slot], sem
            ).wait()

        def reduce(slot):
            zero = jnp.zeros((16,), jnp.float32)

            def row(i, _):
                for c in range(H // 16):
                    sl = pl.ds(c * 16, 16)
                    acc = jnp.maximum(rows_v[slot, i, sl], zero)
                    for j in range(1, MAX_NB):
                        acc = acc + jnp.maximum(rows_v[slot, j * _B + i, sl], zero)
                    out_v[slot, i, sl] = acc
                return 0

            lax.fori_loop(0, _B, row, 0, unroll=False)

        def store(bg, slot, sem):
            pltpu.async_copy(out_v.at[slot], out_hbm.at[pl.ds(bg * _B, _B)], sem)

        def wait_store(slot, sem):
            pltpu.make_async_copy(
                out_v.at[slot], out_hbm.at[pl.ds(0, _B)], sem
            ).wait()

        @pl.when(nb > 0)
        def _():
            fire(base_b, 0, sg0)

        def pair(k, _):
            b0 = base_b + 2 * k
            fire(b0 + 1, 1, sg1)
            wait_gather(0, sg0)

            @pl.when(k > 0)
            def _():
                wait_store(0, so0)

            reduce(0)
            store(b0, 0, so0)

            @pl.when(2 * k + 2 < nb)
            def _():
                fire(b0 + 2, 0, sg0)

            wait_gather(1, sg1)

            @pl.when(k > 0)
            def _():
                wait_store(1, so1)

            reduce(1)
            store(b0 + 1, 1, so1)
            return 0

        lax.fori_loop(0, npairs, pair, 0, unroll=False)

        @pl.when(tail == 1)
        def _():
            wait_gather(0, sg0)

            @pl.when(npairs > 0)
            def _():
                wait_store(0, so0)

            reduce(0)
            store(base_b + nb - 1, 0, so0)

        @pl.when(nb > 0)
        def _():
            wait_store(0, so0)

        @pl.when(npairs > 0)
        def _():
            wait_store(1, so1)

    return gsum(table, idxt)


def kernel(fatoms, fbonds, agraph, bgraph, scope, w_i, w_h, w_o, b_o, w_mh, b_mh, w_mo, b_mo):
    del scope
    d = fatoms.shape[1]

    fb2 = jnp.reshape(fbonds, (N_BONDS // 2, 2 * (d + 16)))
    ni2 = _bond_in(fb2, _block_diag2(w_i))
    ni = jnp.reshape(ni2, (N_BONDS, H))

    nei_sum = _gather_sum_relu_sc(ni, bgraph.T, N_BONDS, 125, 0, 125, 0)
    ns2 = jnp.reshape(nei_sum, (N_BONDS // 2, 2 * H))
    h2 = _bond_update(ns2, _block_diag2(w_h), ni2)
    msg_2 = jnp.reshape(h2, (N_BONDS, H))

    nei_atom = _gather_sum_relu_sc(msg_2, agraph.T, N_ATOMS, 8, 7, 7, 3)
    na2 = jnp.reshape(nei_atom, (N_ATOMS // 2, 2 * H))

    fa2 = jnp.reshape(fatoms, (N_ATOMS // 2, 2 * d))
    ah2 = _atom_out(
        fa2,
        na2,
        _block_diag2(w_o[:d]),
        _block_diag2(w_o[d:]),
        jnp.concatenate([b_o, b_o]).reshape(1, 2 * H),
    )
    atom_h = jnp.reshape(ah2, (N_ATOMS, H))
    mol_o = _mol_head(ah2.reshape(500, 10, 2 * H), w_mh, b_mh, w_mo, b_mo)
    return (atom_h, mol_o)
